# Initial kernel scaffold; baseline (speedup 1.0000x reference)
import jax
import jax.numpy as jnp
from jax.experimental import pallas as pl


def kernel(x, edge_index):
    raise NotImplementedError("write your pallas kernel here")


# speedup vs baseline: 10.5075x; 10.5075x over previous
import functools

import jax
import jax.numpy as jnp
from jax import lax
from jax.experimental import pallas as pl
from jax.experimental.pallas import tpu as pltpu
from jax.experimental.pallas import tpu_sc as plsc


@functools.cache
def _gunpool_sc(B, V, E, d, NC, NS):
    NW = NC * NS
    EW = E // NW
    assert E % NW == 0
    K = 1
    for cand in (128, 120, 112, 104, 96, 88, 80, 72, 64, 56, 48, 40, 32, 24, 16, 8):
        if EW % cand == 0:
            K = cand
            break
    n_chunks = EW // K
    VO = V + E

    mesh = plsc.VectorSubcoreMesh(core_axis_name="c", subcore_axis_name="s")

    @functools.partial(
        pl.kernel,
        out_type=jax.ShapeDtypeStruct((B * VO, d), jnp.float32),
        mesh=mesh,
        scratch_types=[
            pltpu.VMEM((K,), jnp.int32),
            pltpu.VMEM((K,), jnp.int32),
            pltpu.VMEM((K, d), jnp.float32),
            pltpu.VMEM((K, d), jnp.float32),
            pltpu.SemaphoreType.DMA,
            pltpu.SemaphoreType.DMA,
        ],
    )
    def k(x_hbm, src_hbm, dst_hbm, out_hbm,
          idx_s, idx_d, rows_a, rows_b, sem_a, sem_b):
        wid = lax.axis_index("s") * NC + lax.axis_index("c")

        CPY = 320
        n_full = V // CPY
        rem = V - n_full * CPY
        for b in range(B):
            @pl.when(wid < n_full)
            def _():
                r0 = wid * CPY
                pltpu.sync_copy(x_hbm.at[pl.ds(b * V + r0, CPY)],
                                out_hbm.at[pl.ds(b * VO + r0, CPY)])
            if rem:
                @pl.when(wid == n_full)
                def _():
                    r0 = n_full * CPY
                    pltpu.sync_copy(x_hbm.at[pl.ds(b * V + r0, rem)],
                                    out_hbm.at[pl.ds(b * VO + r0, rem)])

        for b in range(B):
            def chunk(c, _, b=b):
                e0 = b * E + wid * EW + c * K
                pltpu.sync_copy(src_hbm.at[pl.ds(e0, K)], idx_s)
                pltpu.sync_copy(dst_hbm.at[pl.ds(e0, K)], idx_d)
                ga = pltpu.async_copy(x_hbm.at[idx_s], rows_a, sem_a)
                gb = pltpu.async_copy(x_hbm.at[idx_d], rows_b, sem_b)
                ga.wait()
                gb.wait()

                def body(i, _):
                    for j in range(d // 16):
                        sl = pl.ds(j * 16, 16)
                        rows_a[i, sl] = (rows_a[i, sl] + rows_b[i, sl]) * 0.5
                    return 0
                lax.fori_loop(0, K, body, 0)

                o0 = b * VO + V + wid * EW + c * K
                pltpu.sync_copy(rows_a, out_hbm.at[pl.ds(o0, K)])
                return 0
            lax.fori_loop(0, n_chunks, chunk, 0)

    return k


def kernel(x, edge_index):
    B, V, d = x.shape
    E = edge_index.shape[1]
    idx = edge_index.astype(jnp.int32)
    offs = (jnp.arange(B, dtype=jnp.int32) * V)[:, None]
    src_all = (idx[0][None, :] + offs).reshape(-1)
    dst_all = (idx[1][None, :] + offs).reshape(-1)
    x2 = x.reshape(B * V, d)
    info = plsc.get_sparse_core_info()
    out = _gunpool_sc(B, V, E, d, info.num_cores, info.num_subcores)(
        x2, src_all, dst_all)
    return out.reshape(B, V + E, d)


# speedup vs baseline: 16.7877x; 1.5977x over previous
import functools

import jax
import jax.numpy as jnp
from jax import lax
from jax.experimental import pallas as pl
from jax.experimental.pallas import tpu as pltpu
from jax.experimental.pallas import tpu_sc as plsc


@functools.cache
def _gunpool_sc(B, V, E, d, NC, NS):
    NW = NC * NS
    EW = E // NW
    assert E % NW == 0
    K = 1
    for cand in (128, 120, 112, 104, 96, 88, 80, 72, 64, 56, 48, 40, 32, 24, 16, 8):
        if EW % cand == 0:
            K = cand
            break
    CPB = EW // K
    N = B * CPB
    assert N >= 4 and N % 2 == 0
    VO = V + E

    mesh = plsc.VectorSubcoreMesh(core_axis_name="c", subcore_axis_name="s")

    @functools.partial(
        pl.kernel,
        out_type=jax.ShapeDtypeStruct((B * VO, d), jnp.float32),
        mesh=mesh,
        scratch_types=[
            pltpu.VMEM((2, K), jnp.int32),
            pltpu.VMEM((2, K), jnp.int32),
            pltpu.VMEM((K, d), jnp.float32),
            pltpu.VMEM((K, d), jnp.float32),
            pltpu.VMEM((K, d), jnp.float32),
            pltpu.VMEM((K, d), jnp.float32),
        ] + [pltpu.SemaphoreType.DMA] * 8,
    )
    def k(x_hbm, idxp_hbm, out_hbm,
          idx0, idx1, ra0, ra1, rb0, rb1,
          is0, is1, gsa0, gsa1, gsb0, gsb1, os0, os1):
        wid = lax.axis_index("s") * NC + lax.axis_index("c")
        idxs, ras, rbs = [idx0, idx1], [ra0, ra1], [rb0, rb1]
        isem, gsa, gsb, osem = [is0, is1], [gsa0, gsa1], [gsb0, gsb1], [os0, os1]

        def split(c):
            if B == 2:
                b = (c >= CPB).astype(jnp.int32) if not isinstance(c, int) \
                    else int(c >= CPB)
            else:
                b = c // CPB
            return b, c - b * CPB

        def fire_idx(c, p):
            b, local = split(c)
            row = b * (E // K) + wid * CPB + local
            pltpu.async_copy(idxp_hbm.at[row], idxs[p], isem[p])

        def drain_idx(p):
            pltpu.make_async_copy(idxp_hbm.at[0], idxs[p], isem[p]).wait()

        def fire_gathers(p):
            pltpu.async_copy(x_hbm.at[idxs[p].at[0]], ras[p], gsa[p])
            pltpu.async_copy(x_hbm.at[idxs[p].at[1]], rbs[p], gsb[p])

        def drain_gathers(p):
            pltpu.make_async_copy(x_hbm.at[idxs[p].at[0]], ras[p], gsa[p]).wait()
            pltpu.make_async_copy(x_hbm.at[idxs[p].at[1]], rbs[p], gsb[p]).wait()

        def compute(p):
            def body(i, _):
                for j in range(d // 16):
                    sl = pl.ds(j * 16, 16)
                    ras[p][i, sl] = (ras[p][i, sl] + rbs[p][i, sl]) * 0.5
                return 0
            lax.fori_loop(0, K, body, 0)

        def fire_out(c, p):
            b, local = split(c)
            o0 = b * VO + V + wid * EW + local * K
            pltpu.async_copy(ras[p], out_hbm.at[pl.ds(o0, K)], osem[p])

        def drain_out(p):
            pltpu.make_async_copy(ras[p], out_hbm.at[pl.ds(0, K)], osem[p]).wait()

        CPY = 320
        n_full = V // CPY
        rem = V - n_full * CPY
        for b in range(B):
            @pl.when(wid < n_full)
            def _():
                r0 = wid * CPY
                pltpu.sync_copy(x_hbm.at[pl.ds(b * V + r0, CPY)],
                                out_hbm.at[pl.ds(b * VO + r0, CPY)])
            if rem:
                @pl.when(wid == n_full)
                def _():
                    r0 = n_full * CPY
                    pltpu.sync_copy(x_hbm.at[pl.ds(b * V + r0, rem)],
                                    out_hbm.at[pl.ds(b * VO + r0, rem)])

        fire_idx(0, 0)
        fire_idx(1, 1)
        drain_idx(0)
        fire_gathers(0)
        drain_gathers(0)
        fire_idx(2, 0)
        drain_idx(1)
        fire_gathers(1)
        compute(0)
        fire_out(0, 0)
        drain_gathers(1)
        fire_idx(3, 1)
        drain_out(0)
        drain_idx(0)
        fire_gathers(0)
        compute(1)
        fire_out(1, 1)

        def step(i, _):
            c0 = 2 + 2 * i
            for p in (0, 1):
                c = c0 + p
                q = 1 - p
                drain_gathers(p)
                fire_idx(c + 2, p)
                drain_out(q)
                drain_idx(q)
                fire_gathers(q)
                compute(p)
                fire_out(c, p)
            return 0
        lax.fori_loop(0, (N - 4) // 2, step, 0)

        drain_gathers(0)
        drain_out(1)
        drain_idx(1)
        fire_gathers(1)
        compute(0)
        fire_out(N - 2, 0)

        drain_gathers(1)
        compute(1)
        fire_out(N - 1, 1)

        drain_out(0)
        drain_out(1)

    return k


def kernel(x, edge_index):
    B, V, d = x.shape
    E = edge_index.shape[1]
    idx = edge_index.astype(jnp.int32)
    offs = (jnp.arange(B, dtype=jnp.int32) * V)[:, None]
    src_all = (idx[0][None, :] + offs).reshape(-1)
    dst_all = (idx[1][None, :] + offs).reshape(-1)
    x2 = x.reshape(B * V, d)
    info = plsc.get_sparse_core_info()
    NC, NS = info.num_cores, info.num_subcores
    NW = NC * NS
    EW = E // NW
    K = 1
    for cand in (128, 120, 112, 104, 96, 88, 80, 72, 64, 56, 48, 40, 32, 24, 16, 8):
        if EW % cand == 0:
            K = cand
            break
    idx_packed = jnp.stack(
        [src_all.reshape(-1, K), dst_all.reshape(-1, K)], axis=1)
    out = _gunpool_sc(B, V, E, d, NC, NS)(x2, idx_packed)
    return out.reshape(B, V + E, d)
